# baseline (speedup 1.0000x reference)
import numpy as np
import jax
import jax.numpy as jnp
from jax.experimental import pallas as pl
from jax.experimental.pallas import tpu as pltpu

_B = 128
_N = 100000
_BLK = 12544
_GRID = -(-_N // _BLK)
_STRIPS = _BLK // 128

_KEY_HI = np.uint32(0)
_KEY_LO = np.uint32(42)
_KS2 = np.uint32(_KEY_HI ^ _KEY_LO ^ np.uint32(0x1BD11BDA))
_ROTS = ((13, 15, 26, 6), (17, 29, 16, 24))
_TINY = np.float32(np.finfo(np.float32).tiny)
_INT_MAX = np.int32(2**31 - 1)
_NEG_INF = np.float32(-np.inf)


def _rotl(x, d):
    return (x << np.uint32(d)) | (x >> np.uint32(32 - d))


def _threefry2x32(x0, x1):
    ks = (_KEY_HI, _KEY_LO, _KS2)
    x0 = x0 + ks[0]
    x1 = x1 + ks[1]
    for i in range(5):
        for r in _ROTS[i % 2]:
            x0 = x0 + x1
            x1 = _rotl(x1, r)
            x1 = x0 ^ x1
        x0 = x0 + ks[(i + 1) % 3]
        x1 = x1 + np.uint32(ks[(i + 2) % 3] + np.uint32(i + 1))
    return x0, x1


def _body(logits_ref, out_ref, best_val):
    j = pl.program_id(0)
    shape = (_B, 128)
    row = jax.lax.broadcasted_iota(jnp.uint32, shape, 0) * np.uint32(_N)
    lane = jax.lax.broadcasted_iota(jnp.int32, shape, 1)

    def strip(k, carry):
        bm, bk = carry
        col0 = k * 128
        gcol = j * _BLK + col0 + lane
        flat = row + gcol.astype(jnp.uint32)
        v0, v1 = _threefry2x32(jnp.zeros(shape, jnp.uint32), flat)
        bits = v0 ^ v1
        float_bits = (bits >> np.uint32(9)) | np.uint32(0x3F800000)
        frac = jax.lax.bitcast_convert_type(float_bits, jnp.float32) - np.float32(1.0)
        u = jnp.maximum(_TINY, frac)
        vals = logits_ref[:, pl.ds(col0, 128)] - jnp.log(-jnp.log(u))
        vals = jnp.where(gcol < _N, vals, _NEG_INF)
        better = vals > bm
        bm = jnp.where(better, vals, bm)
        bk = jnp.where(better, k, bk)
        return bm, bk

    bm, bk = jax.lax.fori_loop(
        0, _STRIPS, strip,
        (jnp.full(shape, _NEG_INF, jnp.float32), jnp.zeros(shape, jnp.int32)),
        unroll=49,
    )

    m = jnp.max(bm, axis=1, keepdims=True)
    cand = j * _BLK + bk * 128 + lane
    idx = jnp.min(jnp.where(bm == m, cand, _INT_MAX), axis=1, keepdims=True)

    @pl.when(j == 0)
    def _():
        best_val[...] = m
        out_ref[...] = idx

    @pl.when(j > 0)
    def _():
        bv = best_val[...]
        better = m > bv
        best_val[...] = jnp.where(better, m, bv)
        out_ref[...] = jnp.where(better, idx, out_ref[...])


def kernel(logits):
    out = pl.pallas_call(
        _body,
        grid=(_GRID,),
        in_specs=[pl.BlockSpec((_B, _BLK), lambda j: (0, j))],
        out_specs=pl.BlockSpec((_B, 1), lambda j: (0, 0)),
        out_shape=jax.ShapeDtypeStruct((_B, 1), jnp.int32),
        scratch_shapes=[pltpu.VMEM((_B, 1), jnp.float32)],
    )(logits)
    return out.reshape(_B)


# speedup vs baseline: 1.0247x; 1.0247x over previous
import numpy as np
import jax
import jax.numpy as jnp
from jax.experimental import pallas as pl
from jax.experimental.pallas import tpu as pltpu

_B = 128
_N = 100000
_BLK = 6272
_GRID = -(-_N // _BLK)
_STRIPS = _BLK // 128

_KEY_HI = np.uint32(0)
_KEY_LO = np.uint32(42)
_KS2 = np.uint32(_KEY_HI ^ _KEY_LO ^ np.uint32(0x1BD11BDA))
_ROTS = ((13, 15, 26, 6), (17, 29, 16, 24))
_TINY = np.float32(np.finfo(np.float32).tiny)
_INT_MAX = np.int32(2**31 - 1)
_NEG_INF = np.float32(-np.inf)


def _rotl(x, d):
    return (x << np.uint32(d)) | (x >> np.uint32(32 - d))


def _threefry2x32(x0, x1):
    ks = (_KEY_HI, _KEY_LO, _KS2)
    x0 = x0 + ks[0]
    x1 = x1 + ks[1]
    for i in range(5):
        for r in _ROTS[i % 2]:
            x0 = x0 + x1
            x1 = _rotl(x1, r)
            x1 = x0 ^ x1
        x0 = x0 + ks[(i + 1) % 3]
        x1 = x1 + np.uint32(ks[(i + 2) % 3] + np.uint32(i + 1))
    return x0, x1


def _body(logits_ref, out_ref, best_val):
    j = pl.program_id(0)
    shape = (_B, 128)
    row = jax.lax.broadcasted_iota(jnp.uint32, shape, 0) * np.uint32(_N)
    lane = jax.lax.broadcasted_iota(jnp.int32, shape, 1)

    def strip(k, carry):
        bm, bk = carry
        col0 = k * 128
        gcol = j * _BLK + col0 + lane
        flat = row + gcol.astype(jnp.uint32)
        v0, v1 = _threefry2x32(jnp.zeros(shape, jnp.uint32), flat)
        bits = v0 ^ v1
        float_bits = (bits >> np.uint32(9)) | np.uint32(0x3F800000)
        frac = jax.lax.bitcast_convert_type(float_bits, jnp.float32) - np.float32(1.0)
        u = jnp.maximum(_TINY, frac)
        vals = logits_ref[:, pl.ds(col0, 128)] - jnp.log(-jnp.log(u))
        vals = jnp.where(gcol < _N, vals, _NEG_INF)
        better = vals > bm
        bm = jnp.where(better, vals, bm)
        bk = jnp.where(better, k, bk)
        return bm, bk

    bm, bk = jax.lax.fori_loop(
        0, _STRIPS, strip,
        (jnp.full(shape, _NEG_INF, jnp.float32), jnp.zeros(shape, jnp.int32)),
        unroll=49,
    )

    m = jnp.max(bm, axis=1, keepdims=True)
    cand = j * _BLK + bk * 128 + lane
    idx = jnp.min(jnp.where(bm == m, cand, _INT_MAX), axis=1, keepdims=True)

    @pl.when(j == 0)
    def _():
        best_val[...] = m
        out_ref[...] = idx

    @pl.when(j > 0)
    def _():
        bv = best_val[...]
        better = m > bv
        best_val[...] = jnp.where(better, m, bv)
        out_ref[...] = jnp.where(better, idx, out_ref[...])


def kernel(logits):
    out = pl.pallas_call(
        _body,
        grid=(_GRID,),
        in_specs=[pl.BlockSpec((_B, _BLK), lambda j: (0, j))],
        out_specs=pl.BlockSpec((_B, 1), lambda j: (0, 0)),
        out_shape=jax.ShapeDtypeStruct((_B, 1), jnp.int32),
        scratch_shapes=[pltpu.VMEM((_B, 1), jnp.float32)],
    )(logits)
    return out.reshape(_B)
